# baseline (speedup 1.0000x reference)
import functools

import jax
import jax.numpy as jnp
from jax.experimental import pallas as pl
from jax.experimental.pallas import tpu as pltpu


def _pack_conv_weights(w, cin, cout, f):
    wr = w.reshape(3, 3, cin, cout)
    zero = jnp.zeros((3, cin, cout), w.dtype)
    rows = []
    for q in list(range(1, f + 1)) + [0, f + 1]:
        cols = []
        for p in range(f):
            dx = q - p
            cols.append(wr[:, dx] if 0 <= dx <= 2 else zero)
        rows.append(jnp.concatenate(cols, axis=2))
    wp = jnp.concatenate(rows, axis=1)
    return wp.reshape(3 * (f + 2) * cin, f * cout)


def _conv_stack_kernel(x_ref, w1, w2, w3, w4, b1, b2, b3, b4, o_ref,
                       acc1, q2, acc2, q3, acc3, q4, acc4):
    bf16 = jnp.bfloat16
    f32 = jnp.float32

    def conv3tap(q, w, b, H, Wq, K):
        return (
            jnp.dot(q[0:H].reshape(H * Wq, K), w[0:K],
                    preferred_element_type=f32)
            + jnp.dot(q[1:H + 1].reshape(H * Wq, K), w[K:2 * K],
                      preferred_element_type=f32)
            + jnp.dot(q[2:H + 2].reshape(H * Wq, K), w[2 * K:3 * K],
                      preferred_element_type=f32)
            + b[...])

    acc1[...] = conv3tap(x_ref[0], w1, b1, 64, 16, 18)

    v = acc1[...].astype(bf16).reshape(64, 16, 128)
    q2[0:1] = jnp.zeros((1, 16, 192), bf16)
    q2[65:66] = jnp.zeros((1, 16, 192), bf16)
    q2[:, 0:1, 128:160] = jnp.zeros((66, 1, 32), bf16)
    q2[:, 15:16, 160:192] = jnp.zeros((66, 1, 32), bf16)
    q2[1:65, :, 0:128] = v
    q2[1:65, 1:16, 128:160] = v[:, 0:15, 96:128]
    q2[1:65, 0:15, 160:192] = v[:, 1:16, 0:32]

    acc2[...] = conv3tap(q2, w2, b2, 64, 16, 192)
    a = acc2[...].reshape(64, 16, 256)
    p_lo = jnp.maximum(a[:, :, 0:64], a[:, :, 64:128])
    p_hi = jnp.maximum(a[:, :, 128:192], a[:, :, 192:256])
    v_lo = jnp.max(p_lo.reshape(32, 2, 16, 64), axis=1).astype(bf16)
    v_hi = jnp.max(p_hi.reshape(32, 2, 16, 64), axis=1).astype(bf16)

    q3[0:1] = jnp.zeros((1, 16, 256), bf16)
    q3[33:34] = jnp.zeros((1, 16, 256), bf16)
    q3[:, 0:1, 128:192] = jnp.zeros((34, 1, 64), bf16)
    q3[:, 15:16, 192:256] = jnp.zeros((34, 1, 64), bf16)
    q3[1:33, :, 0:64] = v_lo
    q3[1:33, :, 64:128] = v_hi
    q3[1:33, 1:16, 128:192] = v_hi[:, 0:15, :]
    q3[1:33, 0:15, 192:256] = v_lo[:, 1:16, :]

    acc3[...] = conv3tap(q3, w3, b3, 32, 16, 256)

    v = acc3[...].astype(bf16).reshape(32, 16, 256)
    q4[0:1] = jnp.zeros((1, 16, 512), bf16)
    q4[33:34] = jnp.zeros((1, 16, 512), bf16)
    q4[:, 0:1, 256:384] = jnp.zeros((34, 1, 128), bf16)
    q4[:, 15:16, 384:512] = jnp.zeros((34, 1, 128), bf16)
    q4[1:33, :, 0:256] = v
    q4[1:33, 1:16, 256:384] = v[:, 0:15, 128:256]
    q4[1:33, 0:15, 384:512] = v[:, 1:16, 0:128]

    acc4[...] = conv3tap(q4, w4, b4, 32, 16, 512)
    a = acc4[...].reshape(32, 16, 256)
    p = jnp.maximum(a[:, :, 0:128], a[:, :, 128:256])
    p = jnp.max(p.reshape(16, 2, 16, 128), axis=1)
    o_ref[0] = p.astype(o_ref.dtype)


def _conv_stack(x_pack, w1, w2, w3, w4, b1, b2, b3, b4):
    N = x_pack.shape[0]
    bf16 = jnp.bfloat16

    flops = 2 * N * (64 * 64 * 32 * 27 + 64 * 64 * 64 * 288
                     + 32 * 32 * 128 * 576 + 32 * 32 * 128 * 1152)
    bytes_accessed = N * (66 * 16 * 18 * 2 + 16 * 16 * 128 * 2)

    return pl.pallas_call(
        _conv_stack_kernel,
        out_shape=jax.ShapeDtypeStruct((N, 16, 16, 128), bf16),
        grid_spec=pltpu.PrefetchScalarGridSpec(
            num_scalar_prefetch=0,
            grid=(N,),
            in_specs=[
                pl.BlockSpec((1, 66, 16, 18), lambda n: (n, 0, 0, 0)),
                pl.BlockSpec((54, 128), lambda n: (0, 0)),
                pl.BlockSpec((576, 256), lambda n: (0, 0)),
                pl.BlockSpec((768, 256), lambda n: (0, 0)),
                pl.BlockSpec((1536, 256), lambda n: (0, 0)),
                pl.BlockSpec((1, 128), lambda n: (0, 0)),
                pl.BlockSpec((1, 256), lambda n: (0, 0)),
                pl.BlockSpec((1, 256), lambda n: (0, 0)),
                pl.BlockSpec((1, 256), lambda n: (0, 0)),
            ],
            out_specs=pl.BlockSpec((1, 16, 16, 128), lambda n: (n, 0, 0, 0)),
            scratch_shapes=[
                pltpu.VMEM((1024, 128), jnp.float32),
                pltpu.VMEM((66, 16, 192), bf16),
                pltpu.VMEM((1024, 256), jnp.float32),
                pltpu.VMEM((34, 16, 256), bf16),
                pltpu.VMEM((512, 256), jnp.float32),
                pltpu.VMEM((34, 16, 512), bf16),
                pltpu.VMEM((512, 256), jnp.float32),
            ],
        ),
        compiler_params=pltpu.CompilerParams(
            dimension_semantics=("parallel",),
            vmem_limit_bytes=32 * 1024 * 1024),
        cost_estimate=pl.CostEstimate(flops=flops, transcendentals=0,
                                      bytes_accessed=bytes_accessed),
    )(x_pack, w1, w2, w3, w4, b1, b2, b3, b4)


def _fc1_kernel(x_ref, w_ref, b_ref, o_ref, acc_ref):
    k = pl.program_id(1)

    @pl.when(k == 0)
    def _init():
        acc_ref[...] = jnp.zeros_like(acc_ref)

    acc_ref[...] += jnp.dot(x_ref[...], w_ref[...],
                            preferred_element_type=jnp.float32)

    @pl.when(k == pl.num_programs(1) - 1)
    def _fin():
        o_ref[...] = jnp.maximum(acc_ref[...] + b_ref[...],
                                 0.0).astype(o_ref.dtype)


def _fc1(feat, w, b):
    M, K = feat.shape
    _, N = w.shape
    tn, tk = N // 2, 2048
    grid = (N // tn, K // tk)

    flops = 2 * M * K * N
    bytes_accessed = M * K * 2 + K * N * 2 + N * 4 + M * N * 2

    return pl.pallas_call(
        _fc1_kernel,
        out_shape=jax.ShapeDtypeStruct((M, N), jnp.bfloat16),
        grid_spec=pltpu.PrefetchScalarGridSpec(
            num_scalar_prefetch=0,
            grid=grid,
            in_specs=[
                pl.BlockSpec((M, tk), lambda j, k: (0, k)),
                pl.BlockSpec((tk, tn), lambda j, k: (k, j)),
                pl.BlockSpec((1, tn), lambda j, k: (0, j)),
            ],
            out_specs=pl.BlockSpec((M, tn), lambda j, k: (0, j)),
            scratch_shapes=[pltpu.VMEM((M, tn), jnp.float32)],
        ),
        compiler_params=pltpu.CompilerParams(
            dimension_semantics=("parallel", "arbitrary"),
            vmem_limit_bytes=32 * 1024 * 1024),
        cost_estimate=pl.CostEstimate(flops=flops, transcendentals=0,
                                      bytes_accessed=bytes_accessed),
    )(feat, w, b.reshape(1, N))


def _fc23_kernel(h_ref, w2_ref, b2_ref, w3_ref, b3_ref, o_ref):
    f32 = jnp.float32
    h2 = jnp.dot(h_ref[...], w2_ref[...], preferred_element_type=f32)
    h2 = jnp.maximum(h2 + b2_ref[...], 0.0).astype(jnp.bfloat16)
    o_ref[...] = jnp.dot(h2, w3_ref[...],
                         preferred_element_type=f32) + b3_ref[...]


def _fc23(h, w2, b2, w3p, b3p):
    M = h.shape[0]
    N2 = w2.shape[1]
    N3 = w3p.shape[1]
    return pl.pallas_call(
        _fc23_kernel,
        out_shape=jax.ShapeDtypeStruct((M, N3), jnp.float32),
        grid_spec=pltpu.PrefetchScalarGridSpec(
            num_scalar_prefetch=0,
            grid=(1,),
            in_specs=[
                pl.BlockSpec((M, 1024), lambda i: (0, 0)),
                pl.BlockSpec((1024, N2), lambda i: (0, 0)),
                pl.BlockSpec((1, N2), lambda i: (0, 0)),
                pl.BlockSpec((512, N3), lambda i: (0, 0)),
                pl.BlockSpec((1, N3), lambda i: (0, 0)),
            ],
            out_specs=pl.BlockSpec((M, N3), lambda i: (0, 0)),
            scratch_shapes=[],
        ),
        compiler_params=pltpu.CompilerParams(
            dimension_semantics=("arbitrary",),
            vmem_limit_bytes=16 * 1024 * 1024),
    )(h, w2, b2.reshape(1, N2), w3p, b3p.reshape(1, N3))


def kernel(x, conv1_w, conv1_b, conv2_w, conv2_b, conv3_w, conv3_b,
           conv4_w, conv4_b, fc1_w, fc1_b, fc2_w, fc2_b, fc3_w, fc3_b):
    bf16 = jnp.bfloat16
    f32 = jnp.float32

    N = x.shape[0]
    x_nhwc = jnp.transpose(x, (0, 2, 3, 1)).astype(bf16)
    base = jnp.pad(x_nhwc.reshape(N, 64, 16, 12),
                   ((0, 0), (1, 1), (0, 0), (0, 0)))
    zc = jnp.zeros((N, 66, 1, 3), bf16)
    left = jnp.concatenate([zc, base[:, :, 0:15, 9:12]], axis=2)
    right = jnp.concatenate([base[:, :, 1:16, 0:3], zc], axis=2)
    x_pack = jnp.concatenate([base, left, right], axis=3)

    w1p = _pack_conv_weights(conv1_w.astype(bf16), 3, 32, 4)
    w2p = _pack_conv_weights(conv2_w.astype(bf16), 32, 64, 4)
    w3p = _pack_conv_weights(conv3_w.astype(bf16), 64, 128, 2)
    w4p = _pack_conv_weights(conv4_w.astype(bf16), 128, 128, 2)
    b1p = jnp.tile(conv1_b.astype(f32), 4).reshape(1, 128)
    b2p = jnp.tile(conv2_b.astype(f32), 4).reshape(1, 256)
    b3p = jnp.tile(conv3_b.astype(f32), 2).reshape(1, 256)
    b4p = jnp.tile(conv4_b.astype(f32), 2).reshape(1, 256)

    out = _conv_stack(x_pack, w1p, w2p, w3p, w4p, b1p, b2p, b3p, b4p)

    feat = out.reshape(out.shape[0], -1)
    h = _fc1(feat, fc1_w.astype(bf16), fc1_b.astype(f32))

    num_classes = fc3_w.shape[1]
    n3p = ((num_classes + 127) // 128) * 128
    fw3p = jnp.pad(fc3_w.astype(bf16), ((0, 0), (0, n3p - num_classes)))
    fb3p = jnp.pad(fc3_b.astype(f32), (0, n3p - num_classes))

    logits = _fc23(h, fc2_w.astype(bf16), fc2_b.astype(f32), fw3p, fb3p)
    return logits[:, :num_classes]


# speedup vs baseline: 1.1422x; 1.1422x over previous
import functools

import jax
import jax.numpy as jnp
from jax.experimental import pallas as pl
from jax.experimental.pallas import tpu as pltpu


def _pack_conv_weights(w, cin, cout, f):
    wr = w.reshape(3, 3, cin, cout)
    zero = jnp.zeros((3, cin, cout), w.dtype)
    rows = []
    for q in list(range(1, f + 1)) + [0, f + 1]:
        cols = []
        for p in range(f):
            dx = q - p
            cols.append(wr[:, dx] if 0 <= dx <= 2 else zero)
        rows.append(jnp.concatenate(cols, axis=2))
    wp = jnp.concatenate(rows, axis=1)
    return wp.reshape(3 * (f + 2) * cin, f * cout)


def _conv_stack_kernel(x_ref, w1, w2, w3, w4, b1, b2, b3, b4, o_ref,
                       q1, acc1, q2, acc2, q3, acc3, q4, acc4):
    bf16 = jnp.bfloat16
    f32 = jnp.float32

    def conv3tap(q, w, b, H, Wq, K):
        return (
            jnp.dot(q[0:H].reshape(H * Wq, K), w[0:K],
                    preferred_element_type=f32)
            + jnp.dot(q[1:H + 1].reshape(H * Wq, K), w[K:2 * K],
                      preferred_element_type=f32)
            + jnp.dot(q[2:H + 2].reshape(H * Wq, K), w[2 * K:3 * K],
                      preferred_element_type=f32)
            + b[...])

    q1[0:1] = jnp.zeros((1, 16, 18), bf16)
    q1[65:66] = jnp.zeros((1, 16, 18), bf16)
    q1[:, 0:1, 12:15] = jnp.zeros((66, 1, 3), bf16)
    q1[:, 15:16, 15:18] = jnp.zeros((66, 1, 3), bf16)
    q1[1:65, :, 0:12] = x_ref[0]
    q1[1:65, 1:16, 12:15] = x_ref[0, :, 0:15, 9:12]
    q1[1:65, 0:15, 15:18] = x_ref[0, :, 1:16, 0:3]
    acc1[...] = conv3tap(q1, w1, b1, 64, 16, 18)

    v = acc1[...].astype(bf16).reshape(64, 16, 128)
    q2[0:1] = jnp.zeros((1, 16, 192), bf16)
    q2[65:66] = jnp.zeros((1, 16, 192), bf16)
    q2[:, 0:1, 128:160] = jnp.zeros((66, 1, 32), bf16)
    q2[:, 15:16, 160:192] = jnp.zeros((66, 1, 32), bf16)
    q2[1:65, :, 0:128] = v
    q2[1:65, 1:16, 128:160] = v[:, 0:15, 96:128]
    q2[1:65, 0:15, 160:192] = v[:, 1:16, 0:32]

    acc2[...] = conv3tap(q2, w2, b2, 64, 16, 192)
    a = acc2[...].reshape(64, 16, 256)
    p_lo = jnp.maximum(a[:, :, 0:64], a[:, :, 64:128])
    p_hi = jnp.maximum(a[:, :, 128:192], a[:, :, 192:256])
    v_lo = jnp.max(p_lo.reshape(32, 2, 16, 64), axis=1).astype(bf16)
    v_hi = jnp.max(p_hi.reshape(32, 2, 16, 64), axis=1).astype(bf16)

    q3[0:1] = jnp.zeros((1, 16, 256), bf16)
    q3[33:34] = jnp.zeros((1, 16, 256), bf16)
    q3[:, 0:1, 128:192] = jnp.zeros((34, 1, 64), bf16)
    q3[:, 15:16, 192:256] = jnp.zeros((34, 1, 64), bf16)
    q3[1:33, :, 0:64] = v_lo
    q3[1:33, :, 64:128] = v_hi
    q3[1:33, 1:16, 128:192] = v_hi[:, 0:15, :]
    q3[1:33, 0:15, 192:256] = v_lo[:, 1:16, :]

    acc3[...] = conv3tap(q3, w3, b3, 32, 16, 256)

    v = acc3[...].astype(bf16).reshape(32, 16, 256)
    q4[0:1] = jnp.zeros((1, 16, 512), bf16)
    q4[33:34] = jnp.zeros((1, 16, 512), bf16)
    q4[:, 0:1, 256:384] = jnp.zeros((34, 1, 128), bf16)
    q4[:, 15:16, 384:512] = jnp.zeros((34, 1, 128), bf16)
    q4[1:33, :, 0:256] = v
    q4[1:33, 1:16, 256:384] = v[:, 0:15, 128:256]
    q4[1:33, 0:15, 384:512] = v[:, 1:16, 0:128]

    acc4[...] = conv3tap(q4, w4, b4, 32, 16, 512)
    a = acc4[...].reshape(32, 16, 256)
    p = jnp.maximum(a[:, :, 0:128], a[:, :, 128:256])
    p = jnp.max(p.reshape(16, 2, 16, 128), axis=1)
    o_ref[0] = p.astype(o_ref.dtype)


def _conv_stack(x_pack, w1, w2, w3, w4, b1, b2, b3, b4):
    N = x_pack.shape[0]
    bf16 = jnp.bfloat16

    flops = 2 * N * (64 * 64 * 32 * 27 + 64 * 64 * 64 * 288
                     + 32 * 32 * 128 * 576 + 32 * 32 * 128 * 1152)
    bytes_accessed = N * (66 * 16 * 18 * 2 + 16 * 16 * 128 * 2)

    return pl.pallas_call(
        _conv_stack_kernel,
        out_shape=jax.ShapeDtypeStruct((N, 16, 16, 128), bf16),
        grid_spec=pltpu.PrefetchScalarGridSpec(
            num_scalar_prefetch=0,
            grid=(N,),
            in_specs=[
                pl.BlockSpec((1, 64, 16, 12), lambda n: (n, 0, 0, 0)),
                pl.BlockSpec((54, 128), lambda n: (0, 0)),
                pl.BlockSpec((576, 256), lambda n: (0, 0)),
                pl.BlockSpec((768, 256), lambda n: (0, 0)),
                pl.BlockSpec((1536, 256), lambda n: (0, 0)),
                pl.BlockSpec((1, 128), lambda n: (0, 0)),
                pl.BlockSpec((1, 256), lambda n: (0, 0)),
                pl.BlockSpec((1, 256), lambda n: (0, 0)),
                pl.BlockSpec((1, 256), lambda n: (0, 0)),
            ],
            out_specs=pl.BlockSpec((1, 16, 16, 128), lambda n: (n, 0, 0, 0)),
            scratch_shapes=[
                pltpu.VMEM((66, 16, 18), bf16),
                pltpu.VMEM((1024, 128), jnp.float32),
                pltpu.VMEM((66, 16, 192), bf16),
                pltpu.VMEM((1024, 256), jnp.float32),
                pltpu.VMEM((34, 16, 256), bf16),
                pltpu.VMEM((512, 256), jnp.float32),
                pltpu.VMEM((34, 16, 512), bf16),
                pltpu.VMEM((512, 256), jnp.float32),
            ],
        ),
        compiler_params=pltpu.CompilerParams(
            dimension_semantics=("parallel",),
            vmem_limit_bytes=32 * 1024 * 1024),
        cost_estimate=pl.CostEstimate(flops=flops, transcendentals=0,
                                      bytes_accessed=bytes_accessed),
    )(x_pack, w1, w2, w3, w4, b1, b2, b3, b4)


def _fc1_kernel(x_ref, w_ref, b_ref, o_ref, acc_ref):
    k = pl.program_id(1)

    @pl.when(k == 0)
    def _init():
        acc_ref[...] = jnp.zeros_like(acc_ref)

    acc_ref[...] += jnp.dot(x_ref[...], w_ref[...],
                            preferred_element_type=jnp.float32)

    @pl.when(k == pl.num_programs(1) - 1)
    def _fin():
        o_ref[...] = jnp.maximum(acc_ref[...] + b_ref[...],
                                 0.0).astype(o_ref.dtype)


def _fc1(feat, w, b):
    M, K = feat.shape
    _, N = w.shape
    tn, tk = N // 2, 2048
    grid = (N // tn, K // tk)

    flops = 2 * M * K * N
    bytes_accessed = M * K * 2 + K * N * 2 + N * 4 + M * N * 2

    return pl.pallas_call(
        _fc1_kernel,
        out_shape=jax.ShapeDtypeStruct((M, N), jnp.bfloat16),
        grid_spec=pltpu.PrefetchScalarGridSpec(
            num_scalar_prefetch=0,
            grid=grid,
            in_specs=[
                pl.BlockSpec((M, tk), lambda j, k: (0, k)),
                pl.BlockSpec((tk, tn), lambda j, k: (k, j)),
                pl.BlockSpec((1, tn), lambda j, k: (0, j)),
            ],
            out_specs=pl.BlockSpec((M, tn), lambda j, k: (0, j)),
            scratch_shapes=[pltpu.VMEM((M, tn), jnp.float32)],
        ),
        compiler_params=pltpu.CompilerParams(
            dimension_semantics=("parallel", "arbitrary"),
            vmem_limit_bytes=32 * 1024 * 1024),
        cost_estimate=pl.CostEstimate(flops=flops, transcendentals=0,
                                      bytes_accessed=bytes_accessed),
    )(feat, w, b.reshape(1, N))


def _fc23_kernel(h_ref, w2_ref, b2_ref, w3_ref, b3_ref, o_ref):
    f32 = jnp.float32
    h2 = jnp.dot(h_ref[...], w2_ref[...], preferred_element_type=f32)
    h2 = jnp.maximum(h2 + b2_ref[...], 0.0).astype(jnp.bfloat16)
    o_ref[...] = jnp.dot(h2, w3_ref[...],
                         preferred_element_type=f32) + b3_ref[...]


def _fc23(h, w2, b2, w3p, b3p):
    M = h.shape[0]
    N2 = w2.shape[1]
    N3 = w3p.shape[1]
    return pl.pallas_call(
        _fc23_kernel,
        out_shape=jax.ShapeDtypeStruct((M, N3), jnp.float32),
        grid_spec=pltpu.PrefetchScalarGridSpec(
            num_scalar_prefetch=0,
            grid=(1,),
            in_specs=[
                pl.BlockSpec((M, 1024), lambda i: (0, 0)),
                pl.BlockSpec((1024, N2), lambda i: (0, 0)),
                pl.BlockSpec((1, N2), lambda i: (0, 0)),
                pl.BlockSpec((512, N3), lambda i: (0, 0)),
                pl.BlockSpec((1, N3), lambda i: (0, 0)),
            ],
            out_specs=pl.BlockSpec((M, N3), lambda i: (0, 0)),
            scratch_shapes=[],
        ),
        compiler_params=pltpu.CompilerParams(
            dimension_semantics=("arbitrary",),
            vmem_limit_bytes=16 * 1024 * 1024),
    )(h, w2, b2.reshape(1, N2), w3p, b3p.reshape(1, N3))


def kernel(x, conv1_w, conv1_b, conv2_w, conv2_b, conv3_w, conv3_b,
           conv4_w, conv4_b, fc1_w, fc1_b, fc2_w, fc2_b, fc3_w, fc3_b):
    bf16 = jnp.bfloat16
    f32 = jnp.float32

    N = x.shape[0]
    x_pack = jnp.transpose(x, (0, 2, 3, 1)).astype(bf16).reshape(N, 64, 16, 12)

    w1p = _pack_conv_weights(conv1_w.astype(bf16), 3, 32, 4)
    w2p = _pack_conv_weights(conv2_w.astype(bf16), 32, 64, 4)
    w3p = _pack_conv_weights(conv3_w.astype(bf16), 64, 128, 2)
    w4p = _pack_conv_weights(conv4_w.astype(bf16), 128, 128, 2)
    b1p = jnp.tile(conv1_b.astype(f32), 4).reshape(1, 128)
    b2p = jnp.tile(conv2_b.astype(f32), 4).reshape(1, 256)
    b3p = jnp.tile(conv3_b.astype(f32), 2).reshape(1, 256)
    b4p = jnp.tile(conv4_b.astype(f32), 2).reshape(1, 256)

    out = _conv_stack(x_pack, w1p, w2p, w3p, w4p, b1p, b2p, b3p, b4p)

    feat = out.reshape(out.shape[0], -1)
    h = _fc1(feat, fc1_w.astype(bf16), fc1_b.astype(f32))

    num_classes = fc3_w.shape[1]
    n3p = ((num_classes + 127) // 128) * 128
    fw3p = jnp.pad(fc3_w.astype(bf16), ((0, 0), (0, n3p - num_classes)))
    fb3p = jnp.pad(fc3_b.astype(f32), (0, n3p - num_classes))

    logits = _fc23(h, fc2_w.astype(bf16), fc2_b.astype(f32), fw3p, fb3p)
    return logits[:, :num_classes]


# speedup vs baseline: 1.3049x; 1.1425x over previous
import functools

import jax
import jax.numpy as jnp
from jax.experimental import pallas as pl
from jax.experimental.pallas import tpu as pltpu


def _pack_conv_weights(w, cin, cout, f):
    wr = w.reshape(3, 3, cin, cout)
    zero = jnp.zeros((3, cin, cout), w.dtype)
    rows = []
    for q in list(range(1, f + 1)) + [0, f + 1]:
        cols = []
        for p in range(f):
            dx = q - p
            cols.append(wr[:, dx] if 0 <= dx <= 2 else zero)
        rows.append(jnp.concatenate(cols, axis=2))
    wp = jnp.concatenate(rows, axis=1)
    return wp.reshape(3 * (f + 2) * cin, f * cout)


def _conv_stack_kernel(x_ref, w1, w2, w3, w4, b1, b2, b3, b4, o_ref,
                       q1, acc1, q2, acc2, q3, acc3, q4, acc4):
    bf16 = jnp.bfloat16
    f32 = jnp.float32

    def conv3tap(q, i, w, b, H, Wq, K):
        return (
            jnp.dot(q[i, 0:H].reshape(H * Wq, K), w[0:K],
                    preferred_element_type=f32)
            + jnp.dot(q[i, 1:H + 1].reshape(H * Wq, K), w[K:2 * K],
                      preferred_element_type=f32)
            + jnp.dot(q[i, 2:H + 2].reshape(H * Wq, K), w[2 * K:3 * K],
                      preferred_element_type=f32)
            + b[...])

    for i in range(2):
        q1[i, 0:1] = jnp.zeros((1, 16, 18), bf16)
        q1[i, 65:66] = jnp.zeros((1, 16, 18), bf16)
        q1[i, :, 0:1, 12:15] = jnp.zeros((66, 1, 3), bf16)
        q1[i, :, 15:16, 15:18] = jnp.zeros((66, 1, 3), bf16)
        q1[i, 1:65, :, 0:12] = x_ref[i]
        q1[i, 1:65, 1:16, 12:15] = x_ref[i, :, 0:15, 9:12]
        q1[i, 1:65, 0:15, 15:18] = x_ref[i, :, 1:16, 0:3]
        acc1[i] = conv3tap(q1, i, w1, b1, 64, 16, 18)

        v = acc1[i].astype(bf16).reshape(64, 16, 128)
        q2[i, 0:1] = jnp.zeros((1, 16, 192), bf16)
        q2[i, 65:66] = jnp.zeros((1, 16, 192), bf16)
        q2[i, :, 0:1, 128:160] = jnp.zeros((66, 1, 32), bf16)
        q2[i, :, 15:16, 160:192] = jnp.zeros((66, 1, 32), bf16)
        q2[i, 1:65, :, 0:128] = v
        q2[i, 1:65, 1:16, 128:160] = v[:, 0:15, 96:128]
        q2[i, 1:65, 0:15, 160:192] = v[:, 1:16, 0:32]

        acc2[i] = conv3tap(q2, i, w2, b2, 64, 16, 192)
        a = acc2[i].reshape(64, 16, 256)
        p_lo = jnp.maximum(a[:, :, 0:64], a[:, :, 64:128])
        p_hi = jnp.maximum(a[:, :, 128:192], a[:, :, 192:256])
        v_lo = jnp.max(p_lo.reshape(32, 2, 16, 64), axis=1).astype(bf16)
        v_hi = jnp.max(p_hi.reshape(32, 2, 16, 64), axis=1).astype(bf16)

        q3[i, 0:1] = jnp.zeros((1, 16, 256), bf16)
        q3[i, 33:34] = jnp.zeros((1, 16, 256), bf16)
        q3[i, :, 0:1, 128:192] = jnp.zeros((34, 1, 64), bf16)
        q3[i, :, 15:16, 192:256] = jnp.zeros((34, 1, 64), bf16)
        q3[i, 1:33, :, 0:64] = v_lo
        q3[i, 1:33, :, 64:128] = v_hi
        q3[i, 1:33, 1:16, 128:192] = v_hi[:, 0:15, :]
        q3[i, 1:33, 0:15, 192:256] = v_lo[:, 1:16, :]

        acc3[i] = conv3tap(q3, i, w3, b3, 32, 16, 256)

        v = acc3[i].astype(bf16).reshape(32, 16, 256)
        q4[i, 0:1] = jnp.zeros((1, 16, 512), bf16)
        q4[i, 33:34] = jnp.zeros((1, 16, 512), bf16)
        q4[i, :, 0:1, 256:384] = jnp.zeros((34, 1, 128), bf16)
        q4[i, :, 15:16, 384:512] = jnp.zeros((34, 1, 128), bf16)
        q4[i, 1:33, :, 0:256] = v
        q4[i, 1:33, 1:16, 256:384] = v[:, 0:15, 128:256]
        q4[i, 1:33, 0:15, 384:512] = v[:, 1:16, 0:128]

        acc4[i] = conv3tap(q4, i, w4, b4, 32, 16, 512)
        a = acc4[i].reshape(32, 16, 256)
        p = jnp.maximum(a[:, :, 0:128], a[:, :, 128:256])
        p = jnp.max(p.reshape(16, 2, 16, 128), axis=1)
        o_ref[i] = p.astype(o_ref.dtype)


def _conv_stack(x_pack, w1, w2, w3, w4, b1, b2, b3, b4):
    N = x_pack.shape[0]
    bf16 = jnp.bfloat16

    flops = 2 * N * (64 * 64 * 32 * 27 + 64 * 64 * 64 * 288
                     + 32 * 32 * 128 * 576 + 32 * 32 * 128 * 1152)
    bytes_accessed = N * (66 * 16 * 18 * 2 + 16 * 16 * 128 * 2)

    return pl.pallas_call(
        _conv_stack_kernel,
        out_shape=jax.ShapeDtypeStruct((N, 16, 16, 128), bf16),
        grid_spec=pltpu.PrefetchScalarGridSpec(
            num_scalar_prefetch=0,
            grid=(N // 2,),
            in_specs=[
                pl.BlockSpec((2, 64, 16, 12), lambda n: (n, 0, 0, 0)),
                pl.BlockSpec((54, 128), lambda n: (0, 0)),
                pl.BlockSpec((576, 256), lambda n: (0, 0)),
                pl.BlockSpec((768, 256), lambda n: (0, 0)),
                pl.BlockSpec((1536, 256), lambda n: (0, 0)),
                pl.BlockSpec((1, 128), lambda n: (0, 0)),
                pl.BlockSpec((1, 256), lambda n: (0, 0)),
                pl.BlockSpec((1, 256), lambda n: (0, 0)),
                pl.BlockSpec((1, 256), lambda n: (0, 0)),
            ],
            out_specs=pl.BlockSpec((2, 16, 16, 128), lambda n: (n, 0, 0, 0)),
            scratch_shapes=[
                pltpu.VMEM((2, 66, 16, 18), bf16),
                pltpu.VMEM((2, 1024, 128), jnp.float32),
                pltpu.VMEM((2, 66, 16, 192), bf16),
                pltpu.VMEM((2, 1024, 256), jnp.float32),
                pltpu.VMEM((2, 34, 16, 256), bf16),
                pltpu.VMEM((2, 512, 256), jnp.float32),
                pltpu.VMEM((2, 34, 16, 512), bf16),
                pltpu.VMEM((2, 512, 256), jnp.float32),
            ],
        ),
        compiler_params=pltpu.CompilerParams(
            dimension_semantics=("parallel",),
            vmem_limit_bytes=32 * 1024 * 1024),
        cost_estimate=pl.CostEstimate(flops=flops, transcendentals=0,
                                      bytes_accessed=bytes_accessed),
    )(x_pack, w1, w2, w3, w4, b1, b2, b3, b4)


def _fc1_kernel(x_ref, w_ref, b_ref, o_ref, acc_ref):
    k = pl.program_id(1)

    @pl.when(k == 0)
    def _init():
        acc_ref[...] = jnp.zeros_like(acc_ref)

    acc_ref[...] += jnp.dot(x_ref[...], w_ref[...],
                            preferred_element_type=jnp.float32)

    @pl.when(k == pl.num_programs(1) - 1)
    def _fin():
        o_ref[...] = jnp.maximum(acc_ref[...] + b_ref[...],
                                 0.0).astype(o_ref.dtype)


def _fc1(feat, w, b):
    M, K = feat.shape
    _, N = w.shape
    tn, tk = N // 2, 2048
    grid = (N // tn, K // tk)

    flops = 2 * M * K * N
    bytes_accessed = M * K * 2 + K * N * 2 + N * 4 + M * N * 2

    return pl.pallas_call(
        _fc1_kernel,
        out_shape=jax.ShapeDtypeStruct((M, N), jnp.bfloat16),
        grid_spec=pltpu.PrefetchScalarGridSpec(
            num_scalar_prefetch=0,
            grid=grid,
            in_specs=[
                pl.BlockSpec((M, tk), lambda j, k: (0, k)),
                pl.BlockSpec((tk, tn), lambda j, k: (k, j)),
                pl.BlockSpec((1, tn), lambda j, k: (0, j)),
            ],
            out_specs=pl.BlockSpec((M, tn), lambda j, k: (0, j)),
            scratch_shapes=[pltpu.VMEM((M, tn), jnp.float32)],
        ),
        compiler_params=pltpu.CompilerParams(
            dimension_semantics=("parallel", "arbitrary"),
            vmem_limit_bytes=32 * 1024 * 1024),
        cost_estimate=pl.CostEstimate(flops=flops, transcendentals=0,
                                      bytes_accessed=bytes_accessed),
    )(feat, w, b.reshape(1, N))


def _fc23_kernel(h_ref, w2_ref, b2_ref, w3_ref, b3_ref, o_ref):
    f32 = jnp.float32
    h2 = jnp.dot(h_ref[...], w2_ref[...], preferred_element_type=f32)
    h2 = jnp.maximum(h2 + b2_ref[...], 0.0).astype(jnp.bfloat16)
    o_ref[...] = jnp.dot(h2, w3_ref[...],
                         preferred_element_type=f32) + b3_ref[...]


def _fc23(h, w2, b2, w3p, b3p):
    M = h.shape[0]
    N2 = w2.shape[1]
    N3 = w3p.shape[1]
    return pl.pallas_call(
        _fc23_kernel,
        out_shape=jax.ShapeDtypeStruct((M, N3), jnp.float32),
        grid_spec=pltpu.PrefetchScalarGridSpec(
            num_scalar_prefetch=0,
            grid=(1,),
            in_specs=[
                pl.BlockSpec((M, 1024), lambda i: (0, 0)),
                pl.BlockSpec((1024, N2), lambda i: (0, 0)),
                pl.BlockSpec((1, N2), lambda i: (0, 0)),
                pl.BlockSpec((512, N3), lambda i: (0, 0)),
                pl.BlockSpec((1, N3), lambda i: (0, 0)),
            ],
            out_specs=pl.BlockSpec((M, N3), lambda i: (0, 0)),
            scratch_shapes=[],
        ),
        compiler_params=pltpu.CompilerParams(
            dimension_semantics=("arbitrary",),
            vmem_limit_bytes=16 * 1024 * 1024),
    )(h, w2, b2.reshape(1, N2), w3p, b3p.reshape(1, N3))


def kernel(x, conv1_w, conv1_b, conv2_w, conv2_b, conv3_w, conv3_b,
           conv4_w, conv4_b, fc1_w, fc1_b, fc2_w, fc2_b, fc3_w, fc3_b):
    bf16 = jnp.bfloat16
    f32 = jnp.float32

    N = x.shape[0]
    x_pack = jnp.transpose(x, (0, 2, 3, 1)).astype(bf16).reshape(N, 64, 16, 12)

    w1p = _pack_conv_weights(conv1_w.astype(bf16), 3, 32, 4)
    w2p = _pack_conv_weights(conv2_w.astype(bf16), 32, 64, 4)
    w3p = _pack_conv_weights(conv3_w.astype(bf16), 64, 128, 2)
    w4p = _pack_conv_weights(conv4_w.astype(bf16), 128, 128, 2)
    b1p = jnp.tile(conv1_b.astype(f32), 4).reshape(1, 128)
    b2p = jnp.tile(conv2_b.astype(f32), 4).reshape(1, 256)
    b3p = jnp.tile(conv3_b.astype(f32), 2).reshape(1, 256)
    b4p = jnp.tile(conv4_b.astype(f32), 2).reshape(1, 256)

    out = _conv_stack(x_pack, w1p, w2p, w3p, w4p, b1p, b2p, b3p, b4p)

    feat = out.reshape(out.shape[0], -1)
    h = _fc1(feat, fc1_w.astype(bf16), fc1_b.astype(f32))

    num_classes = fc3_w.shape[1]
    n3p = ((num_classes + 127) // 128) * 128
    fw3p = jnp.pad(fc3_w.astype(bf16), ((0, 0), (0, n3p - num_classes)))
    fb3p = jnp.pad(fc3_b.astype(f32), (0, n3p - num_classes))

    logits = _fc23(h, fc2_w.astype(bf16), fc2_b.astype(f32), fw3p, fb3p)
    return logits[:, :num_classes]


# speedup vs baseline: 1.4412x; 1.1044x over previous
import functools

import jax
import jax.numpy as jnp
from jax.experimental import pallas as pl
from jax.experimental.pallas import tpu as pltpu


def _pack_conv_weights(w, cin, cout, f):
    wr = w.reshape(3, 3, cin, cout)
    zero = jnp.zeros((3, cin, cout), w.dtype)
    rows = []
    for q in list(range(1, f + 1)) + [0, f + 1]:
        cols = []
        for p in range(f):
            dx = q - p
            cols.append(wr[:, dx] if 0 <= dx <= 2 else zero)
        rows.append(jnp.concatenate(cols, axis=2))
    wp = jnp.concatenate(rows, axis=1)
    return wp.reshape(3 * (f + 2) * cin, f * cout)


def _conv_stack_kernel(x_ref, w1, w2, w3, w4, b1, b2, b3, b4, o_ref,
                       q1, acc1, q2, acc2, q3, acc3, q4, acc4):
    bf16 = jnp.bfloat16
    f32 = jnp.float32

    def conv3tap(q, i, w, b, H, Wq, K):
        return (
            jnp.dot(q[i, 0:H].reshape(H * Wq, K), w[0:K],
                    preferred_element_type=f32)
            + jnp.dot(q[i, 1:H + 1].reshape(H * Wq, K), w[K:2 * K],
                      preferred_element_type=f32)
            + jnp.dot(q[i, 2:H + 2].reshape(H * Wq, K), w[2 * K:3 * K],
                      preferred_element_type=f32)
            + b[...])

    for i in range(4):
        q1[i, 0:1] = jnp.zeros((1, 16, 18), bf16)
        q1[i, 65:66] = jnp.zeros((1, 16, 18), bf16)
        q1[i, :, 0:1, 12:15] = jnp.zeros((66, 1, 3), bf16)
        q1[i, :, 15:16, 15:18] = jnp.zeros((66, 1, 3), bf16)
        q1[i, 1:65, :, 0:12] = x_ref[i]
        q1[i, 1:65, 1:16, 12:15] = x_ref[i, :, 0:15, 9:12]
        q1[i, 1:65, 0:15, 15:18] = x_ref[i, :, 1:16, 0:3]
        acc1[i] = conv3tap(q1, i, w1, b1, 64, 16, 18)

        v = acc1[i].astype(bf16).reshape(64, 16, 128)
        q2[i, 0:1] = jnp.zeros((1, 16, 192), bf16)
        q2[i, 65:66] = jnp.zeros((1, 16, 192), bf16)
        q2[i, :, 0:1, 128:160] = jnp.zeros((66, 1, 32), bf16)
        q2[i, :, 15:16, 160:192] = jnp.zeros((66, 1, 32), bf16)
        q2[i, 1:65, :, 0:128] = v
        q2[i, 1:65, 1:16, 128:160] = v[:, 0:15, 96:128]
        q2[i, 1:65, 0:15, 160:192] = v[:, 1:16, 0:32]

        acc2[i] = conv3tap(q2, i, w2, b2, 64, 16, 192)
        a = acc2[i].reshape(64, 16, 256)
        p_lo = jnp.maximum(a[:, :, 0:64], a[:, :, 64:128])
        p_hi = jnp.maximum(a[:, :, 128:192], a[:, :, 192:256])
        v_lo = jnp.max(p_lo.reshape(32, 2, 16, 64), axis=1).astype(bf16)
        v_hi = jnp.max(p_hi.reshape(32, 2, 16, 64), axis=1).astype(bf16)

        q3[i, 0:1] = jnp.zeros((1, 16, 256), bf16)
        q3[i, 33:34] = jnp.zeros((1, 16, 256), bf16)
        q3[i, :, 0:1, 128:192] = jnp.zeros((34, 1, 64), bf16)
        q3[i, :, 15:16, 192:256] = jnp.zeros((34, 1, 64), bf16)
        q3[i, 1:33, :, 0:64] = v_lo
        q3[i, 1:33, :, 64:128] = v_hi
        q3[i, 1:33, 1:16, 128:192] = v_hi[:, 0:15, :]
        q3[i, 1:33, 0:15, 192:256] = v_lo[:, 1:16, :]

        acc3[i] = conv3tap(q3, i, w3, b3, 32, 16, 256)

        v = acc3[i].astype(bf16).reshape(32, 16, 256)
        q4[i, 0:1] = jnp.zeros((1, 16, 512), bf16)
        q4[i, 33:34] = jnp.zeros((1, 16, 512), bf16)
        q4[i, :, 0:1, 256:384] = jnp.zeros((34, 1, 128), bf16)
        q4[i, :, 15:16, 384:512] = jnp.zeros((34, 1, 128), bf16)
        q4[i, 1:33, :, 0:256] = v
        q4[i, 1:33, 1:16, 256:384] = v[:, 0:15, 128:256]
        q4[i, 1:33, 0:15, 384:512] = v[:, 1:16, 0:128]

        acc4[i] = conv3tap(q4, i, w4, b4, 32, 16, 512)
        a = acc4[i].reshape(32, 16, 256)
        p = jnp.maximum(a[:, :, 0:128], a[:, :, 128:256])
        p = jnp.max(p.reshape(16, 2, 16, 128), axis=1)
        o_ref[i] = p.astype(o_ref.dtype)


def _conv_stack(x_pack, w1, w2, w3, w4, b1, b2, b3, b4):
    N = x_pack.shape[0]
    bf16 = jnp.bfloat16

    flops = 2 * N * (64 * 64 * 32 * 27 + 64 * 64 * 64 * 288
                     + 32 * 32 * 128 * 576 + 32 * 32 * 128 * 1152)
    bytes_accessed = N * (66 * 16 * 18 * 2 + 16 * 16 * 128 * 2)

    return pl.pallas_call(
        _conv_stack_kernel,
        out_shape=jax.ShapeDtypeStruct((N, 16, 16, 128), bf16),
        grid_spec=pltpu.PrefetchScalarGridSpec(
            num_scalar_prefetch=0,
            grid=(N // 4,),
            in_specs=[
                pl.BlockSpec((4, 64, 16, 12), lambda n: (n, 0, 0, 0)),
                pl.BlockSpec((54, 128), lambda n: (0, 0)),
                pl.BlockSpec((576, 256), lambda n: (0, 0)),
                pl.BlockSpec((768, 256), lambda n: (0, 0)),
                pl.BlockSpec((1536, 256), lambda n: (0, 0)),
                pl.BlockSpec((1, 128), lambda n: (0, 0)),
                pl.BlockSpec((1, 256), lambda n: (0, 0)),
                pl.BlockSpec((1, 256), lambda n: (0, 0)),
                pl.BlockSpec((1, 256), lambda n: (0, 0)),
            ],
            out_specs=pl.BlockSpec((4, 16, 16, 128), lambda n: (n, 0, 0, 0)),
            scratch_shapes=[
                pltpu.VMEM((4, 66, 16, 18), bf16),
                pltpu.VMEM((4, 1024, 128), jnp.float32),
                pltpu.VMEM((4, 66, 16, 192), bf16),
                pltpu.VMEM((4, 1024, 256), jnp.float32),
                pltpu.VMEM((4, 34, 16, 256), bf16),
                pltpu.VMEM((4, 512, 256), jnp.float32),
                pltpu.VMEM((4, 34, 16, 512), bf16),
                pltpu.VMEM((4, 512, 256), jnp.float32),
            ],
        ),
        compiler_params=pltpu.CompilerParams(
            dimension_semantics=("parallel",),
            vmem_limit_bytes=32 * 1024 * 1024),
        cost_estimate=pl.CostEstimate(flops=flops, transcendentals=0,
                                      bytes_accessed=bytes_accessed),
    )(x_pack, w1, w2, w3, w4, b1, b2, b3, b4)


def _fc1_kernel(x_ref, w_ref, b_ref, o_ref, acc_ref):
    k = pl.program_id(1)

    @pl.when(k == 0)
    def _init():
        acc_ref[...] = jnp.zeros_like(acc_ref)

    acc_ref[...] += jnp.dot(x_ref[...], w_ref[...],
                            preferred_element_type=jnp.float32)

    @pl.when(k == pl.num_programs(1) - 1)
    def _fin():
        o_ref[...] = jnp.maximum(acc_ref[...] + b_ref[...],
                                 0.0).astype(o_ref.dtype)


def _fc1(feat, w, b):
    M, K = feat.shape
    _, N = w.shape
    tn, tk = N // 2, 2048
    grid = (N // tn, K // tk)

    flops = 2 * M * K * N
    bytes_accessed = M * K * 2 + K * N * 2 + N * 4 + M * N * 2

    return pl.pallas_call(
        _fc1_kernel,
        out_shape=jax.ShapeDtypeStruct((M, N), jnp.bfloat16),
        grid_spec=pltpu.PrefetchScalarGridSpec(
            num_scalar_prefetch=0,
            grid=grid,
            in_specs=[
                pl.BlockSpec((M, tk), lambda j, k: (0, k)),
                pl.BlockSpec((tk, tn), lambda j, k: (k, j)),
                pl.BlockSpec((1, tn), lambda j, k: (0, j)),
            ],
            out_specs=pl.BlockSpec((M, tn), lambda j, k: (0, j)),
            scratch_shapes=[pltpu.VMEM((M, tn), jnp.float32)],
        ),
        compiler_params=pltpu.CompilerParams(
            dimension_semantics=("parallel", "arbitrary"),
            vmem_limit_bytes=32 * 1024 * 1024),
        cost_estimate=pl.CostEstimate(flops=flops, transcendentals=0,
                                      bytes_accessed=bytes_accessed),
    )(feat, w, b.reshape(1, N))


def _fc23_kernel(h_ref, w2_ref, b2_ref, w3_ref, b3_ref, o_ref):
    f32 = jnp.float32
    h2 = jnp.dot(h_ref[...], w2_ref[...], preferred_element_type=f32)
    h2 = jnp.maximum(h2 + b2_ref[...], 0.0).astype(jnp.bfloat16)
    o_ref[...] = jnp.dot(h2, w3_ref[...],
                         preferred_element_type=f32) + b3_ref[...]


def _fc23(h, w2, b2, w3p, b3p):
    M = h.shape[0]
    N2 = w2.shape[1]
    N3 = w3p.shape[1]
    return pl.pallas_call(
        _fc23_kernel,
        out_shape=jax.ShapeDtypeStruct((M, N3), jnp.float32),
        grid_spec=pltpu.PrefetchScalarGridSpec(
            num_scalar_prefetch=0,
            grid=(1,),
            in_specs=[
                pl.BlockSpec((M, 1024), lambda i: (0, 0)),
                pl.BlockSpec((1024, N2), lambda i: (0, 0)),
                pl.BlockSpec((1, N2), lambda i: (0, 0)),
                pl.BlockSpec((512, N3), lambda i: (0, 0)),
                pl.BlockSpec((1, N3), lambda i: (0, 0)),
            ],
            out_specs=pl.BlockSpec((M, N3), lambda i: (0, 0)),
            scratch_shapes=[],
        ),
        compiler_params=pltpu.CompilerParams(
            dimension_semantics=("arbitrary",),
            vmem_limit_bytes=16 * 1024 * 1024),
    )(h, w2, b2.reshape(1, N2), w3p, b3p.reshape(1, N3))


def kernel(x, conv1_w, conv1_b, conv2_w, conv2_b, conv3_w, conv3_b,
           conv4_w, conv4_b, fc1_w, fc1_b, fc2_w, fc2_b, fc3_w, fc3_b):
    bf16 = jnp.bfloat16
    f32 = jnp.float32

    N = x.shape[0]
    x_pack = jnp.transpose(x, (0, 2, 3, 1)).astype(bf16).reshape(N, 64, 16, 12)

    w1p = _pack_conv_weights(conv1_w.astype(bf16), 3, 32, 4)
    w2p = _pack_conv_weights(conv2_w.astype(bf16), 32, 64, 4)
    w3p = _pack_conv_weights(conv3_w.astype(bf16), 64, 128, 2)
    w4p = _pack_conv_weights(conv4_w.astype(bf16), 128, 128, 2)
    b1p = jnp.tile(conv1_b.astype(f32), 4).reshape(1, 128)
    b2p = jnp.tile(conv2_b.astype(f32), 4).reshape(1, 256)
    b3p = jnp.tile(conv3_b.astype(f32), 2).reshape(1, 256)
    b4p = jnp.tile(conv4_b.astype(f32), 2).reshape(1, 256)

    out = _conv_stack(x_pack, w1p, w2p, w3p, w4p, b1p, b2p, b3p, b4p)

    feat = out.reshape(out.shape[0], -1)
    h = _fc1(feat, fc1_w.astype(bf16), fc1_b.astype(f32))

    num_classes = fc3_w.shape[1]
    n3p = ((num_classes + 127) // 128) * 128
    fw3p = jnp.pad(fc3_w.astype(bf16), ((0, 0), (0, n3p - num_classes)))
    fb3p = jnp.pad(fc3_b.astype(f32), (0, n3p - num_classes))

    logits = _fc23(h, fc2_w.astype(bf16), fc2_b.astype(f32), fw3p, fb3p)
    return logits[:, :num_classes]


# speedup vs baseline: 1.5277x; 1.0600x over previous
import functools

import jax
import jax.numpy as jnp
from jax.experimental import pallas as pl
from jax.experimental.pallas import tpu as pltpu


def _pack_conv_weights(w, cin, cout, f):
    wr = w.reshape(3, 3, cin, cout)
    zero = jnp.zeros((3, cin, cout), w.dtype)
    rows = []
    for q in list(range(1, f + 1)) + [0, f + 1]:
        cols = []
        for p in range(f):
            dx = q - p
            cols.append(wr[:, dx] if 0 <= dx <= 2 else zero)
        rows.append(jnp.concatenate(cols, axis=2))
    wp = jnp.concatenate(rows, axis=1)
    return wp.reshape(3 * (f + 2) * cin, f * cout)


def _conv_stack_kernel(x_ref, w1, w2, w3, w4, b1, b2, b3, b4, o_ref,
                       q1, acc1, q2, acc2, q3, acc3, q4, acc4):
    bf16 = jnp.bfloat16
    f32 = jnp.float32

    def conv3tap(q, i, w, b, H, Wq, K):
        return (
            jnp.dot(q[i, 0:H].reshape(H * Wq, K), w[0:K],
                    preferred_element_type=f32)
            + jnp.dot(q[i, 1:H + 1].reshape(H * Wq, K), w[K:2 * K],
                      preferred_element_type=f32)
            + jnp.dot(q[i, 2:H + 2].reshape(H * Wq, K), w[2 * K:3 * K],
                      preferred_element_type=f32)
            + b[...])

    for i in range(8):
        q1[i, 0:1] = jnp.zeros((1, 16, 18), bf16)
        q1[i, 65:66] = jnp.zeros((1, 16, 18), bf16)
        q1[i, :, 0:1, 12:15] = jnp.zeros((66, 1, 3), bf16)
        q1[i, :, 15:16, 15:18] = jnp.zeros((66, 1, 3), bf16)
        q1[i, 1:65, :, 0:12] = x_ref[i]
        q1[i, 1:65, 1:16, 12:15] = x_ref[i, :, 0:15, 9:12]
        q1[i, 1:65, 0:15, 15:18] = x_ref[i, :, 1:16, 0:3]
        acc1[i] = conv3tap(q1, i, w1, b1, 64, 16, 18)

        v = acc1[i].astype(bf16).reshape(64, 16, 128)
        q2[i, 0:1] = jnp.zeros((1, 16, 192), bf16)
        q2[i, 65:66] = jnp.zeros((1, 16, 192), bf16)
        q2[i, :, 0:1, 128:160] = jnp.zeros((66, 1, 32), bf16)
        q2[i, :, 15:16, 160:192] = jnp.zeros((66, 1, 32), bf16)
        q2[i, 1:65, :, 0:128] = v
        q2[i, 1:65, 1:16, 128:160] = v[:, 0:15, 96:128]
        q2[i, 1:65, 0:15, 160:192] = v[:, 1:16, 0:32]

        acc2[i] = conv3tap(q2, i, w2, b2, 64, 16, 192)
        a = acc2[i].reshape(64, 16, 256)
        p_lo = jnp.maximum(a[:, :, 0:64], a[:, :, 64:128])
        p_hi = jnp.maximum(a[:, :, 128:192], a[:, :, 192:256])
        v_lo = jnp.max(p_lo.reshape(32, 2, 16, 64), axis=1).astype(bf16)
        v_hi = jnp.max(p_hi.reshape(32, 2, 16, 64), axis=1).astype(bf16)

        q3[i, 0:1] = jnp.zeros((1, 16, 256), bf16)
        q3[i, 33:34] = jnp.zeros((1, 16, 256), bf16)
        q3[i, :, 0:1, 128:192] = jnp.zeros((34, 1, 64), bf16)
        q3[i, :, 15:16, 192:256] = jnp.zeros((34, 1, 64), bf16)
        q3[i, 1:33, :, 0:64] = v_lo
        q3[i, 1:33, :, 64:128] = v_hi
        q3[i, 1:33, 1:16, 128:192] = v_hi[:, 0:15, :]
        q3[i, 1:33, 0:15, 192:256] = v_lo[:, 1:16, :]

        acc3[i] = conv3tap(q3, i, w3, b3, 32, 16, 256)

        v = acc3[i].astype(bf16).reshape(32, 16, 256)
        q4[i, 0:1] = jnp.zeros((1, 16, 512), bf16)
        q4[i, 33:34] = jnp.zeros((1, 16, 512), bf16)
        q4[i, :, 0:1, 256:384] = jnp.zeros((34, 1, 128), bf16)
        q4[i, :, 15:16, 384:512] = jnp.zeros((34, 1, 128), bf16)
        q4[i, 1:33, :, 0:256] = v
        q4[i, 1:33, 1:16, 256:384] = v[:, 0:15, 128:256]
        q4[i, 1:33, 0:15, 384:512] = v[:, 1:16, 0:128]

        acc4[i] = conv3tap(q4, i, w4, b4, 32, 16, 512)
        a = acc4[i].reshape(32, 16, 256)
        p = jnp.maximum(a[:, :, 0:128], a[:, :, 128:256])
        p = jnp.max(p.reshape(16, 2, 16, 128), axis=1)
        o_ref[i] = p.astype(o_ref.dtype)


def _conv_stack(x_pack, w1, w2, w3, w4, b1, b2, b3, b4):
    N = x_pack.shape[0]
    bf16 = jnp.bfloat16

    flops = 2 * N * (64 * 64 * 32 * 27 + 64 * 64 * 64 * 288
                     + 32 * 32 * 128 * 576 + 32 * 32 * 128 * 1152)
    bytes_accessed = N * (66 * 16 * 18 * 2 + 16 * 16 * 128 * 2)

    return pl.pallas_call(
        _conv_stack_kernel,
        out_shape=jax.ShapeDtypeStruct((N, 16, 16, 128), bf16),
        grid_spec=pltpu.PrefetchScalarGridSpec(
            num_scalar_prefetch=0,
            grid=(N // 8,),
            in_specs=[
                pl.BlockSpec((8, 64, 16, 12), lambda n: (n, 0, 0, 0)),
                pl.BlockSpec((54, 128), lambda n: (0, 0)),
                pl.BlockSpec((576, 256), lambda n: (0, 0)),
                pl.BlockSpec((768, 256), lambda n: (0, 0)),
                pl.BlockSpec((1536, 256), lambda n: (0, 0)),
                pl.BlockSpec((1, 128), lambda n: (0, 0)),
                pl.BlockSpec((1, 256), lambda n: (0, 0)),
                pl.BlockSpec((1, 256), lambda n: (0, 0)),
                pl.BlockSpec((1, 256), lambda n: (0, 0)),
            ],
            out_specs=pl.BlockSpec((8, 16, 16, 128), lambda n: (n, 0, 0, 0)),
            scratch_shapes=[
                pltpu.VMEM((8, 66, 16, 18), bf16),
                pltpu.VMEM((8, 1024, 128), jnp.float32),
                pltpu.VMEM((8, 66, 16, 192), bf16),
                pltpu.VMEM((8, 1024, 256), jnp.float32),
                pltpu.VMEM((8, 34, 16, 256), bf16),
                pltpu.VMEM((8, 512, 256), jnp.float32),
                pltpu.VMEM((8, 34, 16, 512), bf16),
                pltpu.VMEM((8, 512, 256), jnp.float32),
            ],
        ),
        compiler_params=pltpu.CompilerParams(
            dimension_semantics=("parallel",),
            vmem_limit_bytes=48 * 1024 * 1024),
        cost_estimate=pl.CostEstimate(flops=flops, transcendentals=0,
                                      bytes_accessed=bytes_accessed),
    )(x_pack, w1, w2, w3, w4, b1, b2, b3, b4)


def _fc1_kernel(x_ref, w_ref, b_ref, o_ref, acc_ref):
    k = pl.program_id(1)

    @pl.when(k == 0)
    def _init():
        acc_ref[...] = jnp.zeros_like(acc_ref)

    acc_ref[...] += jnp.dot(x_ref[...], w_ref[...],
                            preferred_element_type=jnp.float32)

    @pl.when(k == pl.num_programs(1) - 1)
    def _fin():
        o_ref[...] = jnp.maximum(acc_ref[...] + b_ref[...],
                                 0.0).astype(o_ref.dtype)


def _fc1(feat, w, b):
    M, K = feat.shape
    _, N = w.shape
    tn, tk = N // 2, 2048
    grid = (N // tn, K // tk)

    flops = 2 * M * K * N
    bytes_accessed = M * K * 2 + K * N * 2 + N * 4 + M * N * 2

    return pl.pallas_call(
        _fc1_kernel,
        out_shape=jax.ShapeDtypeStruct((M, N), jnp.bfloat16),
        grid_spec=pltpu.PrefetchScalarGridSpec(
            num_scalar_prefetch=0,
            grid=grid,
            in_specs=[
                pl.BlockSpec((M, tk), lambda j, k: (0, k)),
                pl.BlockSpec((tk, tn), lambda j, k: (k, j)),
                pl.BlockSpec((1, tn), lambda j, k: (0, j)),
            ],
            out_specs=pl.BlockSpec((M, tn), lambda j, k: (0, j)),
            scratch_shapes=[pltpu.VMEM((M, tn), jnp.float32)],
        ),
        compiler_params=pltpu.CompilerParams(
            dimension_semantics=("parallel", "arbitrary"),
            vmem_limit_bytes=32 * 1024 * 1024),
        cost_estimate=pl.CostEstimate(flops=flops, transcendentals=0,
                                      bytes_accessed=bytes_accessed),
    )(feat, w, b.reshape(1, N))


def _fc23_kernel(h_ref, w2_ref, b2_ref, w3_ref, b3_ref, o_ref):
    f32 = jnp.float32
    h2 = jnp.dot(h_ref[...], w2_ref[...], preferred_element_type=f32)
    h2 = jnp.maximum(h2 + b2_ref[...], 0.0).astype(jnp.bfloat16)
    o_ref[...] = jnp.dot(h2, w3_ref[...],
                         preferred_element_type=f32) + b3_ref[...]


def _fc23(h, w2, b2, w3p, b3p):
    M = h.shape[0]
    N2 = w2.shape[1]
    N3 = w3p.shape[1]
    return pl.pallas_call(
        _fc23_kernel,
        out_shape=jax.ShapeDtypeStruct((M, N3), jnp.float32),
        grid_spec=pltpu.PrefetchScalarGridSpec(
            num_scalar_prefetch=0,
            grid=(1,),
            in_specs=[
                pl.BlockSpec((M, 1024), lambda i: (0, 0)),
                pl.BlockSpec((1024, N2), lambda i: (0, 0)),
                pl.BlockSpec((1, N2), lambda i: (0, 0)),
                pl.BlockSpec((512, N3), lambda i: (0, 0)),
                pl.BlockSpec((1, N3), lambda i: (0, 0)),
            ],
            out_specs=pl.BlockSpec((M, N3), lambda i: (0, 0)),
            scratch_shapes=[],
        ),
        compiler_params=pltpu.CompilerParams(
            dimension_semantics=("arbitrary",),
            vmem_limit_bytes=16 * 1024 * 1024),
    )(h, w2, b2.reshape(1, N2), w3p, b3p.reshape(1, N3))


def kernel(x, conv1_w, conv1_b, conv2_w, conv2_b, conv3_w, conv3_b,
           conv4_w, conv4_b, fc1_w, fc1_b, fc2_w, fc2_b, fc3_w, fc3_b):
    bf16 = jnp.bfloat16
    f32 = jnp.float32

    N = x.shape[0]
    x_pack = jnp.transpose(x, (0, 2, 3, 1)).astype(bf16).reshape(N, 64, 16, 12)

    w1p = _pack_conv_weights(conv1_w.astype(bf16), 3, 32, 4)
    w2p = _pack_conv_weights(conv2_w.astype(bf16), 32, 64, 4)
    w3p = _pack_conv_weights(conv3_w.astype(bf16), 64, 128, 2)
    w4p = _pack_conv_weights(conv4_w.astype(bf16), 128, 128, 2)
    b1p = jnp.tile(conv1_b.astype(f32), 4).reshape(1, 128)
    b2p = jnp.tile(conv2_b.astype(f32), 4).reshape(1, 256)
    b3p = jnp.tile(conv3_b.astype(f32), 2).reshape(1, 256)
    b4p = jnp.tile(conv4_b.astype(f32), 2).reshape(1, 256)

    out = _conv_stack(x_pack, w1p, w2p, w3p, w4p, b1p, b2p, b3p, b4p)

    feat = out.reshape(out.shape[0], -1)
    h = _fc1(feat, fc1_w.astype(bf16), fc1_b.astype(f32))

    num_classes = fc3_w.shape[1]
    n3p = ((num_classes + 127) // 128) * 128
    fw3p = jnp.pad(fc3_w.astype(bf16), ((0, 0), (0, n3p - num_classes)))
    fb3p = jnp.pad(fc3_b.astype(f32), (0, n3p - num_classes))

    logits = _fc23(h, fc2_w.astype(bf16), fc2_b.astype(f32), fw3p, fb3p)
    return logits[:, :num_classes]


# speedup vs baseline: 1.5697x; 1.0275x over previous
import functools

import jax
import jax.numpy as jnp
from jax.experimental import pallas as pl
from jax.experimental.pallas import tpu as pltpu


def _pack_conv_weights(w, cin, cout, f):
    wr = w.reshape(3, 3, cin, cout)
    zero = jnp.zeros((3, cin, cout), w.dtype)
    rows = []
    for q in list(range(1, f + 1)) + [0, f + 1]:
        cols = []
        for p in range(f):
            dx = q - p
            cols.append(wr[:, dx] if 0 <= dx <= 2 else zero)
        rows.append(jnp.concatenate(cols, axis=2))
    wp = jnp.concatenate(rows, axis=1)
    return wp.reshape(3 * (f + 2) * cin, f * cout)


def _conv_stack_kernel(x_ref, w1, w2, w3, w4, b1, b2, b3, b4, o_ref,
                       q1, acc1, q2, acc2, q3, acc3, q4, acc4):
    bf16 = jnp.bfloat16
    f32 = jnp.float32

    def conv3tap(q, i, w, b, H, Wq, K):
        return (
            jnp.dot(q[i, 0:H].reshape(H * Wq, K), w[0:K],
                    preferred_element_type=f32)
            + jnp.dot(q[i, 1:H + 1].reshape(H * Wq, K), w[K:2 * K],
                      preferred_element_type=f32)
            + jnp.dot(q[i, 2:H + 2].reshape(H * Wq, K), w[2 * K:3 * K],
                      preferred_element_type=f32)
            + b[...])

    for i in range(8):
        q1[i, 0:1] = jnp.zeros((1, 16, 18), bf16)
        q1[i, 65:66] = jnp.zeros((1, 16, 18), bf16)
        q1[i, :, 0:1, 12:15] = jnp.zeros((66, 1, 3), bf16)
        q1[i, :, 15:16, 15:18] = jnp.zeros((66, 1, 3), bf16)
        q1[i, 1:65, :, 0:12] = x_ref[i]
        q1[i, 1:65, 1:16, 12:15] = x_ref[i, :, 0:15, 9:12]
        q1[i, 1:65, 0:15, 15:18] = x_ref[i, :, 1:16, 0:3]
        acc1[i] = conv3tap(q1, i, w1, b1, 64, 16, 18)

        v = acc1[i].astype(bf16).reshape(64, 16, 128)
        q2[i, 0:1] = jnp.zeros((1, 16, 192), bf16)
        q2[i, 65:66] = jnp.zeros((1, 16, 192), bf16)
        q2[i, :, 0:1, 128:160] = jnp.zeros((66, 1, 32), bf16)
        q2[i, :, 15:16, 160:192] = jnp.zeros((66, 1, 32), bf16)
        q2[i, 1:65, :, 0:128] = v
        q2[i, 1:65, 1:16, 128:160] = v[:, 0:15, 96:128]
        q2[i, 1:65, 0:15, 160:192] = v[:, 1:16, 0:32]

        acc2[i] = conv3tap(q2, i, w2, b2, 64, 16, 192)
        a = acc2[i].reshape(64, 16, 256)
        p_lo = jnp.maximum(a[:, :, 0:64], a[:, :, 64:128])
        p_hi = jnp.maximum(a[:, :, 128:192], a[:, :, 192:256])
        v_lo = jnp.max(p_lo.reshape(32, 2, 16, 64), axis=1).astype(bf16)
        v_hi = jnp.max(p_hi.reshape(32, 2, 16, 64), axis=1).astype(bf16)

        q3[i, 0:1] = jnp.zeros((1, 16, 256), bf16)
        q3[i, 33:34] = jnp.zeros((1, 16, 256), bf16)
        q3[i, :, 0:1, 128:192] = jnp.zeros((34, 1, 64), bf16)
        q3[i, :, 15:16, 192:256] = jnp.zeros((34, 1, 64), bf16)
        q3[i, 1:33, :, 0:64] = v_lo
        q3[i, 1:33, :, 64:128] = v_hi
        q3[i, 1:33, 1:16, 128:192] = v_hi[:, 0:15, :]
        q3[i, 1:33, 0:15, 192:256] = v_lo[:, 1:16, :]

        acc3[i] = conv3tap(q3, i, w3, b3, 32, 16, 256)

        v = acc3[i].astype(bf16).reshape(32, 16, 256)
        q4[i, 0:1] = jnp.zeros((1, 16, 512), bf16)
        q4[i, 33:34] = jnp.zeros((1, 16, 512), bf16)
        q4[i, :, 0:1, 256:384] = jnp.zeros((34, 1, 128), bf16)
        q4[i, :, 15:16, 384:512] = jnp.zeros((34, 1, 128), bf16)
        q4[i, 1:33, :, 0:256] = v
        q4[i, 1:33, 1:16, 256:384] = v[:, 0:15, 128:256]
        q4[i, 1:33, 0:15, 384:512] = v[:, 1:16, 0:128]

        acc4[i] = conv3tap(q4, i, w4, b4, 32, 16, 512)
        a = acc4[i].reshape(32, 16, 256)
        p = jnp.maximum(a[:, :, 0:128], a[:, :, 128:256])
        p = jnp.max(p.reshape(16, 2, 16, 128), axis=1)
        o_ref[i] = p.astype(o_ref.dtype)


def _conv_stack(x_pack, w1, w2, w3, w4, b1, b2, b3, b4):
    N = x_pack.shape[0]
    bf16 = jnp.bfloat16

    flops = 2 * N * (64 * 64 * 32 * 27 + 64 * 64 * 64 * 288
                     + 32 * 32 * 128 * 576 + 32 * 32 * 128 * 1152)
    bytes_accessed = N * (66 * 16 * 18 * 2 + 16 * 16 * 128 * 2)

    return pl.pallas_call(
        _conv_stack_kernel,
        out_shape=jax.ShapeDtypeStruct((N, 16, 16, 128), bf16),
        grid_spec=pltpu.PrefetchScalarGridSpec(
            num_scalar_prefetch=0,
            grid=(N // 8,),
            in_specs=[
                pl.BlockSpec((8, 64, 16, 12), lambda n: (n, 0, 0, 0)),
                pl.BlockSpec((54, 128), lambda n: (0, 0)),
                pl.BlockSpec((576, 256), lambda n: (0, 0)),
                pl.BlockSpec((768, 256), lambda n: (0, 0)),
                pl.BlockSpec((1536, 256), lambda n: (0, 0)),
                pl.BlockSpec((1, 128), lambda n: (0, 0)),
                pl.BlockSpec((1, 256), lambda n: (0, 0)),
                pl.BlockSpec((1, 256), lambda n: (0, 0)),
                pl.BlockSpec((1, 256), lambda n: (0, 0)),
            ],
            out_specs=pl.BlockSpec((8, 16, 16, 128), lambda n: (n, 0, 0, 0)),
            scratch_shapes=[
                pltpu.VMEM((8, 66, 16, 18), bf16),
                pltpu.VMEM((8, 1024, 128), jnp.float32),
                pltpu.VMEM((8, 66, 16, 192), bf16),
                pltpu.VMEM((8, 1024, 256), jnp.float32),
                pltpu.VMEM((8, 34, 16, 256), bf16),
                pltpu.VMEM((8, 512, 256), jnp.float32),
                pltpu.VMEM((8, 34, 16, 512), bf16),
                pltpu.VMEM((8, 512, 256), jnp.float32),
            ],
        ),
        compiler_params=pltpu.CompilerParams(
            dimension_semantics=("parallel",),
            vmem_limit_bytes=48 * 1024 * 1024),
        cost_estimate=pl.CostEstimate(flops=flops, transcendentals=0,
                                      bytes_accessed=bytes_accessed),
    )(x_pack, w1, w2, w3, w4, b1, b2, b3, b4)


def _fc1_kernel(x_ref, w_ref, b_ref, o_ref, acc_ref, *, tk):
    k = pl.program_id(1)

    @pl.when(k == 0)
    def _init():
        acc_ref[...] = jnp.zeros_like(acc_ref)

    acc_ref[...] += jnp.dot(x_ref[:, pl.ds(k * tk, tk)], w_ref[...],
                            preferred_element_type=jnp.float32)

    @pl.when(k == pl.num_programs(1) - 1)
    def _fin():
        o_ref[...] = jnp.maximum(acc_ref[...] + b_ref[...],
                                 0.0).astype(o_ref.dtype)


def _fc1(feat, w, b):
    M, K = feat.shape
    _, N = w.shape
    tn, tk = N // 2, 4096
    grid = (N // tn, K // tk)

    flops = 2 * M * K * N
    bytes_accessed = M * K * 2 + K * N * 2 + N * 4 + M * N * 2

    return pl.pallas_call(
        functools.partial(_fc1_kernel, tk=tk),
        out_shape=jax.ShapeDtypeStruct((M, N), jnp.bfloat16),
        grid_spec=pltpu.PrefetchScalarGridSpec(
            num_scalar_prefetch=0,
            grid=grid,
            in_specs=[
                pl.BlockSpec((M, K), lambda j, k: (0, 0)),
                pl.BlockSpec((tk, tn), lambda j, k: (k, j)),
                pl.BlockSpec((1, tn), lambda j, k: (0, j)),
            ],
            out_specs=pl.BlockSpec((M, tn), lambda j, k: (0, j)),
            scratch_shapes=[pltpu.VMEM((M, tn), jnp.float32)],
        ),
        compiler_params=pltpu.CompilerParams(
            dimension_semantics=("parallel", "arbitrary"),
            vmem_limit_bytes=32 * 1024 * 1024),
        cost_estimate=pl.CostEstimate(flops=flops, transcendentals=0,
                                      bytes_accessed=bytes_accessed),
    )(feat, w, b.reshape(1, N))


def _fc23_kernel(h_ref, w2_ref, b2_ref, w3_ref, b3_ref, o_ref):
    f32 = jnp.float32
    h2 = jnp.dot(h_ref[...], w2_ref[...], preferred_element_type=f32)
    h2 = jnp.maximum(h2 + b2_ref[...], 0.0).astype(jnp.bfloat16)
    o_ref[...] = jnp.dot(h2, w3_ref[...],
                         preferred_element_type=f32) + b3_ref[...]


def _fc23(h, w2, b2, w3p, b3p):
    M = h.shape[0]
    N2 = w2.shape[1]
    N3 = w3p.shape[1]
    return pl.pallas_call(
        _fc23_kernel,
        out_shape=jax.ShapeDtypeStruct((M, N3), jnp.float32),
        grid_spec=pltpu.PrefetchScalarGridSpec(
            num_scalar_prefetch=0,
            grid=(1,),
            in_specs=[
                pl.BlockSpec((M, 1024), lambda i: (0, 0)),
                pl.BlockSpec((1024, N2), lambda i: (0, 0)),
                pl.BlockSpec((1, N2), lambda i: (0, 0)),
                pl.BlockSpec((512, N3), lambda i: (0, 0)),
                pl.BlockSpec((1, N3), lambda i: (0, 0)),
            ],
            out_specs=pl.BlockSpec((M, N3), lambda i: (0, 0)),
            scratch_shapes=[],
        ),
        compiler_params=pltpu.CompilerParams(
            dimension_semantics=("arbitrary",),
            vmem_limit_bytes=16 * 1024 * 1024),
    )(h, w2, b2.reshape(1, N2), w3p, b3p.reshape(1, N3))


def kernel(x, conv1_w, conv1_b, conv2_w, conv2_b, conv3_w, conv3_b,
           conv4_w, conv4_b, fc1_w, fc1_b, fc2_w, fc2_b, fc3_w, fc3_b):
    bf16 = jnp.bfloat16
    f32 = jnp.float32

    N = x.shape[0]
    x_pack = jnp.transpose(x, (0, 2, 3, 1)).astype(bf16).reshape(N, 64, 16, 12)

    w1p = _pack_conv_weights(conv1_w.astype(bf16), 3, 32, 4)
    w2p = _pack_conv_weights(conv2_w.astype(bf16), 32, 64, 4)
    w3p = _pack_conv_weights(conv3_w.astype(bf16), 64, 128, 2)
    w4p = _pack_conv_weights(conv4_w.astype(bf16), 128, 128, 2)
    b1p = jnp.tile(conv1_b.astype(f32), 4).reshape(1, 128)
    b2p = jnp.tile(conv2_b.astype(f32), 4).reshape(1, 256)
    b3p = jnp.tile(conv3_b.astype(f32), 2).reshape(1, 256)
    b4p = jnp.tile(conv4_b.astype(f32), 2).reshape(1, 256)

    out = _conv_stack(x_pack, w1p, w2p, w3p, w4p, b1p, b2p, b3p, b4p)

    feat = out.reshape(out.shape[0], -1)
    h = _fc1(feat, fc1_w.astype(bf16), fc1_b.astype(f32))

    num_classes = fc3_w.shape[1]
    n3p = ((num_classes + 127) // 128) * 128
    fw3p = jnp.pad(fc3_w.astype(bf16), ((0, 0), (0, n3p - num_classes)))
    fb3p = jnp.pad(fc3_b.astype(f32), (0, n3p - num_classes))

    logits = _fc23(h, fc2_w.astype(bf16), fc2_b.astype(f32), fw3p, fb3p)
    return logits[:, :num_classes]


# speedup vs baseline: 1.5941x; 1.0155x over previous
import functools

import jax
import jax.numpy as jnp
from jax.experimental import pallas as pl
from jax.experimental.pallas import tpu as pltpu


def _pack_conv_weights(w, cin, cout, f):
    wr = w.reshape(3, 3, cin, cout)
    zero = jnp.zeros((3, cin, cout), w.dtype)
    rows = []
    for q in list(range(1, f + 1)) + [0, f + 1]:
        cols = []
        for p in range(f):
            dx = q - p
            cols.append(wr[:, dx] if 0 <= dx <= 2 else zero)
        rows.append(jnp.concatenate(cols, axis=2))
    wp = jnp.concatenate(rows, axis=1)
    return wp.reshape(3 * (f + 2) * cin, f * cout)


def _conv_stack_kernel(x_ref, w1, w2, w3, w4, b1, b2, b3, b4, o_ref,
                       q1, acc1, q2, acc2, q3, acc3, q4, acc4):
    bf16 = jnp.bfloat16
    f32 = jnp.float32

    def conv3tap(q, i, w, b, H, Wq, K):
        return (
            jnp.dot(q[i, 0:H].reshape(H * Wq, K), w[0:K],
                    preferred_element_type=f32)
            + jnp.dot(q[i, 1:H + 1].reshape(H * Wq, K), w[K:2 * K],
                      preferred_element_type=f32)
            + jnp.dot(q[i, 2:H + 2].reshape(H * Wq, K), w[2 * K:3 * K],
                      preferred_element_type=f32)
            + b[...])

    for i in range(8):
        q1[i, 0:1] = jnp.zeros((1, 16, 18), bf16)
        q1[i, 65:66] = jnp.zeros((1, 16, 18), bf16)
        q1[i, :, 0:1, 12:15] = jnp.zeros((66, 1, 3), bf16)
        q1[i, :, 15:16, 15:18] = jnp.zeros((66, 1, 3), bf16)
        q1[i, 1:65, :, 0:12] = x_ref[i]
        q1[i, 1:65, 1:16, 12:15] = x_ref[i, :, 0:15, 9:12]
        q1[i, 1:65, 0:15, 15:18] = x_ref[i, :, 1:16, 0:3]
        acc1[i] = conv3tap(q1, i, w1, b1, 64, 16, 18)

        v = acc1[i].astype(bf16).reshape(64, 16, 128)
        q2[i, 0:1] = jnp.zeros((1, 16, 192), bf16)
        q2[i, 65:66] = jnp.zeros((1, 16, 192), bf16)
        q2[i, :, 0:1, 128:160] = jnp.zeros((66, 1, 32), bf16)
        q2[i, :, 15:16, 160:192] = jnp.zeros((66, 1, 32), bf16)
        q2[i, 1:65, :, 0:128] = v
        q2[i, 1:65, 1:16, 128:160] = v[:, 0:15, 96:128]
        q2[i, 1:65, 0:15, 160:192] = v[:, 1:16, 0:32]

        acc2[i] = conv3tap(q2, i, w2, b2, 64, 16, 192)
        a = acc2[i].reshape(64, 16, 256)
        p_lo = jnp.maximum(a[:, :, 0:64], a[:, :, 64:128])
        p_hi = jnp.maximum(a[:, :, 128:192], a[:, :, 192:256])
        v_lo = jnp.max(p_lo.reshape(32, 2, 16, 64), axis=1).astype(bf16)
        v_hi = jnp.max(p_hi.reshape(32, 2, 16, 64), axis=1).astype(bf16)

        q3[i, 0:1] = jnp.zeros((1, 16, 256), bf16)
        q3[i, 33:34] = jnp.zeros((1, 16, 256), bf16)
        q3[i, :, 0:1, 128:192] = jnp.zeros((34, 1, 64), bf16)
        q3[i, :, 15:16, 192:256] = jnp.zeros((34, 1, 64), bf16)
        q3[i, 1:33, :, 0:64] = v_lo
        q3[i, 1:33, :, 64:128] = v_hi
        q3[i, 1:33, 1:16, 128:192] = v_hi[:, 0:15, :]
        q3[i, 1:33, 0:15, 192:256] = v_lo[:, 1:16, :]

        acc3[i] = conv3tap(q3, i, w3, b3, 32, 16, 256)

        v = acc3[i].astype(bf16).reshape(32, 16, 256)
        q4[i, 0:1] = jnp.zeros((1, 16, 512), bf16)
        q4[i, 33:34] = jnp.zeros((1, 16, 512), bf16)
        q4[i, :, 0:1, 256:384] = jnp.zeros((34, 1, 128), bf16)
        q4[i, :, 15:16, 384:512] = jnp.zeros((34, 1, 128), bf16)
        q4[i, 1:33, :, 0:256] = v
        q4[i, 1:33, 1:16, 256:384] = v[:, 0:15, 128:256]
        q4[i, 1:33, 0:15, 384:512] = v[:, 1:16, 0:128]

        acc4[i] = conv3tap(q4, i, w4, b4, 32, 16, 512)
        a = acc4[i].reshape(32, 16, 256)
        p = jnp.maximum(a[:, :, 0:128], a[:, :, 128:256])
        p = jnp.max(p.reshape(16, 2, 16, 128), axis=1)
        o_ref[i] = p.astype(o_ref.dtype)


def _conv_stack(x_pack, w1, w2, w3, w4, b1, b2, b3, b4):
    N = x_pack.shape[0]
    bf16 = jnp.bfloat16

    flops = 2 * N * (64 * 64 * 32 * 27 + 64 * 64 * 64 * 288
                     + 32 * 32 * 128 * 576 + 32 * 32 * 128 * 1152)
    bytes_accessed = N * (66 * 16 * 18 * 2 + 16 * 16 * 128 * 2)

    return pl.pallas_call(
        _conv_stack_kernel,
        out_shape=jax.ShapeDtypeStruct((N, 16, 16, 128), bf16),
        grid_spec=pltpu.PrefetchScalarGridSpec(
            num_scalar_prefetch=0,
            grid=(N // 8,),
            in_specs=[
                pl.BlockSpec((8, 64, 16, 12), lambda n: (n, 0, 0, 0)),
                pl.BlockSpec((54, 128), lambda n: (0, 0)),
                pl.BlockSpec((576, 256), lambda n: (0, 0)),
                pl.BlockSpec((768, 256), lambda n: (0, 0)),
                pl.BlockSpec((1536, 256), lambda n: (0, 0)),
                pl.BlockSpec((1, 128), lambda n: (0, 0)),
                pl.BlockSpec((1, 256), lambda n: (0, 0)),
                pl.BlockSpec((1, 256), lambda n: (0, 0)),
                pl.BlockSpec((1, 256), lambda n: (0, 0)),
            ],
            out_specs=pl.BlockSpec((8, 16, 16, 128), lambda n: (n, 0, 0, 0)),
            scratch_shapes=[
                pltpu.VMEM((8, 66, 16, 18), bf16),
                pltpu.VMEM((8, 1024, 128), jnp.float32),
                pltpu.VMEM((8, 66, 16, 192), bf16),
                pltpu.VMEM((8, 1024, 256), jnp.float32),
                pltpu.VMEM((8, 34, 16, 256), bf16),
                pltpu.VMEM((8, 512, 256), jnp.float32),
                pltpu.VMEM((8, 34, 16, 512), bf16),
                pltpu.VMEM((8, 512, 256), jnp.float32),
            ],
        ),
        compiler_params=pltpu.CompilerParams(
            dimension_semantics=("parallel",),
            vmem_limit_bytes=48 * 1024 * 1024),
        cost_estimate=pl.CostEstimate(flops=flops, transcendentals=0,
                                      bytes_accessed=bytes_accessed),
    )(x_pack, w1, w2, w3, w4, b1, b2, b3, b4)


def _fc1_kernel(x_ref, w_ref, b_ref, o_ref, acc_ref, *, tk):
    k = pl.program_id(1)

    @pl.when(k == 0)
    def _init():
        acc_ref[...] = jnp.zeros_like(acc_ref)

    acc_ref[...] += jnp.dot(x_ref[:, pl.ds(k * tk, tk)], w_ref[...],
                            preferred_element_type=jnp.float32)

    @pl.when(k == pl.num_programs(1) - 1)
    def _fin():
        o_ref[...] = jnp.maximum(acc_ref[...] + b_ref[...],
                                 0.0).astype(o_ref.dtype)


def _fc1(feat, w, b):
    M, K = feat.shape
    _, N = w.shape
    tn, tk = N // 2, 4096
    grid = (N // tn, K // tk)

    flops = 2 * M * K * N
    bytes_accessed = M * K * 2 + K * N * 2 + N * 4 + M * N * 2

    return pl.pallas_call(
        functools.partial(_fc1_kernel, tk=tk),
        out_shape=jax.ShapeDtypeStruct((M, N), jnp.bfloat16),
        grid_spec=pltpu.PrefetchScalarGridSpec(
            num_scalar_prefetch=0,
            grid=grid,
            in_specs=[
                pl.BlockSpec((M, K), lambda j, k: (0, 0)),
                pl.BlockSpec((tk, tn), lambda j, k: (k, j)),
                pl.BlockSpec((1, tn), lambda j, k: (0, j)),
            ],
            out_specs=pl.BlockSpec((M, tn), lambda j, k: (0, j)),
            scratch_shapes=[pltpu.VMEM((M, tn), jnp.float32)],
        ),
        compiler_params=pltpu.CompilerParams(
            dimension_semantics=("parallel", "arbitrary"),
            vmem_limit_bytes=32 * 1024 * 1024),
        cost_estimate=pl.CostEstimate(flops=flops, transcendentals=0,
                                      bytes_accessed=bytes_accessed),
    )(feat, w, b.reshape(1, N))


def _fc23_kernel(h_ref, w2_ref, b2_ref, w3_ref, b3_ref, o_ref):
    f32 = jnp.float32
    h2 = jnp.dot(h_ref[...], w2_ref[...], preferred_element_type=f32)
    h2 = jnp.maximum(h2 + b2_ref[...], 0.0).astype(jnp.bfloat16)
    o_ref[...] = jnp.dot(h2, w3_ref[...],
                         preferred_element_type=f32) + b3_ref[...]


def _fc23(h, w2, b2, w3p, b3p):
    M = h.shape[0]
    N2 = w2.shape[1]
    N3 = w3p.shape[1]
    return pl.pallas_call(
        _fc23_kernel,
        out_shape=jax.ShapeDtypeStruct((M, N3), jnp.float32),
        grid_spec=pltpu.PrefetchScalarGridSpec(
            num_scalar_prefetch=0,
            grid=(1,),
            in_specs=[
                pl.BlockSpec((M, 1024), lambda i: (0, 0)),
                pl.BlockSpec((1024, N2), lambda i: (0, 0)),
                pl.BlockSpec((1, N2), lambda i: (0, 0)),
                pl.BlockSpec((512, N3), lambda i: (0, 0)),
                pl.BlockSpec((1, N3), lambda i: (0, 0)),
            ],
            out_specs=pl.BlockSpec((M, N3), lambda i: (0, 0)),
            scratch_shapes=[],
        ),
        compiler_params=pltpu.CompilerParams(
            dimension_semantics=("arbitrary",),
            vmem_limit_bytes=16 * 1024 * 1024),
    )(h, w2, b2.reshape(1, N2), w3p, b3p.reshape(1, N3))


def kernel(x, conv1_w, conv1_b, conv2_w, conv2_b, conv3_w, conv3_b,
           conv4_w, conv4_b, fc1_w, fc1_b, fc2_w, fc2_b, fc3_w, fc3_b):
    bf16 = jnp.bfloat16
    f32 = jnp.float32

    N = x.shape[0]
    x_pack = jnp.swapaxes(x.astype(bf16).reshape(N, 3, 4096), 1,
                          2).reshape(N, 64, 16, 12)

    w1p = _pack_conv_weights(conv1_w.astype(bf16), 3, 32, 4)
    w2p = _pack_conv_weights(conv2_w.astype(bf16), 32, 64, 4)
    w3p = _pack_conv_weights(conv3_w.astype(bf16), 64, 128, 2)
    w4p = _pack_conv_weights(conv4_w.astype(bf16), 128, 128, 2)
    b1p = jnp.tile(conv1_b.astype(f32), 4).reshape(1, 128)
    b2p = jnp.tile(conv2_b.astype(f32), 4).reshape(1, 256)
    b3p = jnp.tile(conv3_b.astype(f32), 2).reshape(1, 256)
    b4p = jnp.tile(conv4_b.astype(f32), 2).reshape(1, 256)

    out = _conv_stack(x_pack, w1p, w2p, w3p, w4p, b1p, b2p, b3p, b4p)

    feat = out.reshape(out.shape[0], -1)
    h = _fc1(feat, fc1_w.astype(bf16), fc1_b.astype(f32))

    num_classes = fc3_w.shape[1]
    n3p = ((num_classes + 127) // 128) * 128
    fw3p = jnp.pad(fc3_w.astype(bf16), ((0, 0), (0, n3p - num_classes)))
    fb3p = jnp.pad(fc3_b.astype(f32), (0, n3p - num_classes))

    logits = _fc23(h, fc2_w.astype(bf16), fc2_b.astype(f32), fw3p, fb3p)
    return logits[:, :num_classes]
